# baseline (speedup 1.0000x reference)
import functools

import jax
import jax.numpy as jnp
from jax import lax
from jax.experimental import pallas as pl
from jax.experimental.pallas import tpu as pltpu
from jax.experimental.pallas import tpu_sc as plsc

C = 19
N = 8 * 384 * 384
B = 1024
NW = 32
CH = N // NW
WBLK = 18432
NHB = (384 * 384) // WBLK


def _softmax_errs_body(lg_ref, lab_ref, errs_ref):
    x = lg_ref[0].reshape(C, WBLK)
    lab = lab_ref[0].reshape(1, WBLK)
    m = jnp.max(x, axis=0, keepdims=True)
    ex = jnp.exp(x - m)
    p = ex / jnp.sum(ex, axis=0, keepdims=True)
    oh = lax.broadcasted_iota(jnp.int32, (C, WBLK), 0) == lab
    errs_ref[...] = jnp.where(oh, 1.0 - p, p)


def _softmax_errs(logits, label):
    hrows = 384 // NHB
    return pl.pallas_call(
        _softmax_errs_body,
        grid=(8, NHB),
        in_specs=[
            pl.BlockSpec((1, C, hrows, 384), lambda n, hb: (n, 0, hb, 0)),
            pl.BlockSpec((1, hrows, 384), lambda n, hb: (n, hb, 0)),
        ],
        out_specs=pl.BlockSpec((C, WBLK), lambda n, hb: (0, n * NHB + hb)),
        out_shape=jax.ShapeDtypeStruct((C, N), jnp.float32),
    )(logits, label)


def _hist_body(errs_hbm, lab_hbm, pcnt_hbm,
               lab_v, err_a, err_b, cnt_v, sem_a, sem_b):
    wid = lax.axis_index("s") * 2 + lax.axis_index("c")
    base = wid * CH
    pltpu.sync_copy(lab_hbm.at[pl.ds(base, CH)], lab_v)
    ones = jnp.full((16,), 1, jnp.int32)
    zeros_i = jnp.zeros((16,), jnp.int32)

    def dma(c, buf, sem):
        return pltpu.make_async_copy(
            errs_hbm.at[pl.ds(c, 1), pl.ds(base, CH)], buf, sem)

    def work(c, buf):
        @plsc.parallel_loop(0, 2 * B, 16, unroll=8)
        def zero_body(i):
            cnt_v[pl.ds(i, 16)] = zeros_i

        @plsc.parallel_loop(0, CH, 16, unroll=32)
        def pix_body(i):
            e = buf[0, pl.ds(i, 16)]
            lb = lab_v[pl.ds(i, 16)]
            b = jnp.minimum((e * float(B)).astype(jnp.int32), B - 1)
            idx = b + jnp.where(lb == c, B, 0)
            plsc.addupdate_scatter(cnt_v, [idx], ones)

        off = (c * NW + wid) * (2 * B)
        pltpu.sync_copy(cnt_v, pcnt_hbm.at[pl.ds(off, 2 * B)])

    dma(0, err_a, sem_a).start()

    def pair_body(j, _):
        c0 = 2 * j
        dma(c0, err_a, sem_a).wait()
        dma(c0 + 1, err_b, sem_b).start()
        work(c0, err_a)
        dma(c0 + 1, err_b, sem_b).wait()

        @pl.when(c0 + 2 < C)
        def _():
            dma(c0 + 2, err_a, sem_a).start()

        work(c0 + 1, err_b)
        return 0

    lax.fori_loop(0, C // 2, pair_body, 0)
    dma(C - 1, err_a, sem_a).wait()
    work(C - 1, err_a)


def _histograms(errs2d, lab_flat):
    mesh = plsc.VectorSubcoreMesh(core_axis_name="c", subcore_axis_name="s", num_cores=2, num_subcores=16)
    return pl.kernel(
        _hist_body,
        out_type=jax.ShapeDtypeStruct((C * NW * 2 * B,), jnp.int32),
        mesh=mesh,
        scratch_types=[
            pltpu.VMEM((CH,), jnp.int32),
            pltpu.VMEM((1, CH), jnp.float32),
            pltpu.VMEM((1, CH), jnp.float32),
            pltpu.VMEM((2 * B,), jnp.int32),
            pltpu.SemaphoreType.DMA,
            pltpu.SemaphoreType.DMA,
        ],
        compiler_params=pltpu.CompilerParams(needs_layout_passes=False),
    )(errs2d, lab_flat)


def _loss_body(pcnt_hbm, out_hbm,
               acc_c, buf_c, out_v):
    wid = lax.axis_index("s") * 2 + lax.axis_index("c")

    @pl.when(wid < C)
    def _():
        c = wid
        pltpu.sync_copy(pcnt_hbm.at[pl.ds(c * NW * 2 * B, NW * 2 * B)], buf_c)

        def wsum_body(i, _):
            sl = pl.ds(i * 16, 16)
            v = buf_c[sl]

            def inner(w, v):
                return v + buf_c[pl.ds(w * 2 * B + i * 16, 16)]
            acc_c[sl] = lax.fori_loop(1, NW, inner, v)
            return 0
        lax.fori_loop(0, (2 * B) // 16, wsum_body, 0)

        def pos_body(j, p):
            return p + jnp.sum(acc_c[pl.ds(B + j * 16, 16)])
        P = lax.fori_loop(0, B // 16, pos_body, jnp.int32(0))
        Pf = P.astype(jnp.float32)

        def jacc(m, k):
            den = Pf + m - k
            return jnp.where(den > 0.0, 1.0 - (Pf - k) / jnp.where(den > 0.0, den, 1.0), 0.0)

        lanes_f = lax.iota(jnp.int32, 16).astype(jnp.float32)

        def blk_body(j, carry):
            m0, k0, acc = carry
            b0 = B - 16 * (j + 1)
            cn = lax.rev(acc_c[pl.ds(b0, 16)], (0,))
            cp = lax.rev(acc_c[pl.ds(B + b0, 16)], (0,))
            n = cn + cp
            mc = jnp.cumsum(n) + m0
            kc = jnp.cumsum(cp) + k0
            mcf = mc.astype(jnp.float32)
            kcf = kc.astype(jnp.float32)
            nf = n.astype(jnp.float32)
            kf = cp.astype(jnp.float32)
            j_after = jacc(mcf, kcf)
            j_before = jacc(mcf - nf, kcf - kf)
            mean = (jnp.float32(b0) + 15.5 - lanes_f) * jnp.float32(1.0 / B)
            acc = acc + jnp.sum(mean * (j_after - j_before))
            return (m0 + jnp.sum(n), k0 + jnp.sum(cp), acc)

        _, _, loss = lax.fori_loop(
            0, B // 16, blk_body,
            (jnp.int32(0), jnp.int32(0), jnp.float32(0.0)))

        out_v[...] = jnp.broadcast_to(loss, (16,))
        pltpu.sync_copy(out_v, out_hbm.at[pl.ds(c * 16, 16)])


def _losses(pcnt):
    mesh = plsc.VectorSubcoreMesh(core_axis_name="c", subcore_axis_name="s", num_cores=2, num_subcores=16)
    return pl.kernel(
        _loss_body,
        out_type=jax.ShapeDtypeStruct((C * 16,), jnp.float32),
        mesh=mesh,
        scratch_types=[
            pltpu.VMEM((2 * B,), jnp.int32),
            pltpu.VMEM((NW * 2 * B,), jnp.int32),
            pltpu.VMEM((16,), jnp.float32),
        ],
        compiler_params=pltpu.CompilerParams(needs_layout_passes=False),
    )(pcnt)


def kernel(logits, label):
    errs = _softmax_errs(logits, label)
    lab_flat = label.reshape(-1)
    pcnt = _histograms(errs, lab_flat)
    lossv = _losses(pcnt)
    loss = jnp.mean(lossv.reshape(C, 16)[:, 0])
    return (loss, errs)


# speedup vs baseline: 1.3774x; 1.3774x over previous
import functools

import jax
import jax.numpy as jnp
from jax import lax
from jax.experimental import pallas as pl
from jax.experimental.pallas import tpu as pltpu
from jax.experimental.pallas import tpu_sc as plsc

C = 19
N = 8 * 384 * 384
B = 1024
NW = 32
CH = N // NW
WBLK = 36864
NHB = (384 * 384) // WBLK


def _softmax_errs_body(lg_ref, lab_ref, errs_ref):
    x = lg_ref[0].reshape(C, WBLK)
    lab = lab_ref[0].reshape(1, WBLK)
    m = jnp.max(x, axis=0, keepdims=True)
    ex = jnp.exp(x - m)
    p = ex / jnp.sum(ex, axis=0, keepdims=True)
    oh = lax.broadcasted_iota(jnp.int32, (C, WBLK), 0) == lab
    errs_ref[...] = jnp.where(oh, 1.0 - p, p)


def _softmax_errs(logits, label):
    hrows = 384 // NHB
    return pl.pallas_call(
        _softmax_errs_body,
        grid=(8, NHB),
        in_specs=[
            pl.BlockSpec((1, C, hrows, 384), lambda n, hb: (n, 0, hb, 0)),
            pl.BlockSpec((1, hrows, 384), lambda n, hb: (n, hb, 0)),
        ],
        out_specs=pl.BlockSpec((C, WBLK), lambda n, hb: (0, n * NHB + hb)),
        out_shape=jax.ShapeDtypeStruct((C, N), jnp.float32),
    )(logits, label)


def _hist_body(errs_hbm, lab_hbm, pcnt_hbm,
               lab_v, err_a, err_b, cnt_v, sem_a, sem_b):
    wid = lax.axis_index("s") * 2 + lax.axis_index("c")
    base = wid * CH
    pltpu.sync_copy(lab_hbm.at[pl.ds(base, CH)], lab_v)
    ones = jnp.full((16,), 1, jnp.int32)
    zeros_i = jnp.zeros((16,), jnp.int32)

    def dma(c, buf, sem):
        return pltpu.make_async_copy(
            errs_hbm.at[pl.ds(c, 1), pl.ds(base, CH)], buf, sem)

    def work(c, buf):
        @plsc.parallel_loop(0, 2 * B, 16, unroll=8)
        def zero_body(i):
            cnt_v[pl.ds(i, 16)] = zeros_i

        @plsc.parallel_loop(0, CH, 16, unroll=16)
        def pix_body(i):
            e = buf[0, pl.ds(i, 16)]
            lb = lab_v[pl.ds(i, 16)]
            b = jnp.minimum((e * float(B)).astype(jnp.int32), B - 1)
            idx = b + jnp.where(lb == c, B, 0)
            plsc.addupdate_scatter(cnt_v, [idx], ones)

        off = (c * NW + wid) * (2 * B)
        pltpu.sync_copy(cnt_v, pcnt_hbm.at[pl.ds(off, 2 * B)])

    dma(0, err_a, sem_a).start()

    def pair_body(j, _):
        c0 = 2 * j
        dma(c0, err_a, sem_a).wait()
        dma(c0 + 1, err_b, sem_b).start()
        work(c0, err_a)
        dma(c0 + 1, err_b, sem_b).wait()

        @pl.when(c0 + 2 < C)
        def _():
            dma(c0 + 2, err_a, sem_a).start()

        work(c0 + 1, err_b)
        return 0

    lax.fori_loop(0, C // 2, pair_body, 0)
    dma(C - 1, err_a, sem_a).wait()
    work(C - 1, err_a)


def _histograms(errs2d, lab_flat):
    mesh = plsc.VectorSubcoreMesh(core_axis_name="c", subcore_axis_name="s", num_cores=2, num_subcores=16)
    return pl.kernel(
        _hist_body,
        out_type=jax.ShapeDtypeStruct((C * NW * 2 * B,), jnp.int32),
        mesh=mesh,
        scratch_types=[
            pltpu.VMEM((CH,), jnp.int32),
            pltpu.VMEM((1, CH), jnp.float32),
            pltpu.VMEM((1, CH), jnp.float32),
            pltpu.VMEM((2 * B,), jnp.int32),
            pltpu.SemaphoreType.DMA,
            pltpu.SemaphoreType.DMA,
        ],
        compiler_params=pltpu.CompilerParams(needs_layout_passes=False),
    )(errs2d, lab_flat)


def _loss_body(pcnt_hbm, out_hbm,
               acc_c, buf_c, out_v):
    wid = lax.axis_index("s") * 2 + lax.axis_index("c")

    @pl.when(wid < C)
    def _():
        c = wid
        pltpu.sync_copy(pcnt_hbm.at[pl.ds(c * NW * 2 * B, NW * 2 * B)], buf_c)

        def wsum_body(i, _):
            sl = pl.ds(i * 16, 16)
            v = buf_c[sl]

            def inner(w, v):
                return v + buf_c[pl.ds(w * 2 * B + i * 16, 16)]
            acc_c[sl] = lax.fori_loop(1, NW, inner, v)
            return 0
        lax.fori_loop(0, (2 * B) // 16, wsum_body, 0)

        def pos_body(j, p):
            return p + jnp.sum(acc_c[pl.ds(B + j * 16, 16)])
        P = lax.fori_loop(0, B // 16, pos_body, jnp.int32(0))
        Pf = P.astype(jnp.float32)

        def jacc(m, k):
            den = Pf + m - k
            return jnp.where(den > 0.0, 1.0 - (Pf - k) / jnp.where(den > 0.0, den, 1.0), 0.0)

        lanes_f = lax.iota(jnp.int32, 16).astype(jnp.float32)

        def blk_body(j, carry):
            m0, k0, acc = carry
            b0 = B - 16 * (j + 1)
            cn = lax.rev(acc_c[pl.ds(b0, 16)], (0,))
            cp = lax.rev(acc_c[pl.ds(B + b0, 16)], (0,))
            n = cn + cp
            mc = jnp.cumsum(n) + m0
            kc = jnp.cumsum(cp) + k0
            mcf = mc.astype(jnp.float32)
            kcf = kc.astype(jnp.float32)
            nf = n.astype(jnp.float32)
            kf = cp.astype(jnp.float32)
            j_after = jacc(mcf, kcf)
            j_before = jacc(mcf - nf, kcf - kf)
            mean = (jnp.float32(b0) + 15.5 - lanes_f) * jnp.float32(1.0 / B)
            acc = acc + jnp.sum(mean * (j_after - j_before))
            return (m0 + jnp.sum(n), k0 + jnp.sum(cp), acc)

        _, _, loss = lax.fori_loop(
            0, B // 16, blk_body,
            (jnp.int32(0), jnp.int32(0), jnp.float32(0.0)))

        out_v[...] = jnp.broadcast_to(loss, (16,))
        pltpu.sync_copy(out_v, out_hbm.at[pl.ds(c * 16, 16)])


def _losses(pcnt):
    mesh = plsc.VectorSubcoreMesh(core_axis_name="c", subcore_axis_name="s", num_cores=2, num_subcores=16)
    return pl.kernel(
        _loss_body,
        out_type=jax.ShapeDtypeStruct((C * 16,), jnp.float32),
        mesh=mesh,
        scratch_types=[
            pltpu.VMEM((2 * B,), jnp.int32),
            pltpu.VMEM((NW * 2 * B,), jnp.int32),
            pltpu.VMEM((16,), jnp.float32),
        ],
        compiler_params=pltpu.CompilerParams(needs_layout_passes=False),
    )(pcnt)


def kernel(logits, label):
    errs = _softmax_errs(logits, label)
    lab_flat = label.reshape(-1)
    pcnt = _histograms(errs, lab_flat)
    lossv = _losses(pcnt)
    loss = jnp.mean(lossv.reshape(C, 16)[:, 0])
    return (loss, errs)


# speedup vs baseline: 1.4282x; 1.0369x over previous
import functools

import jax
import jax.numpy as jnp
from jax import lax
from jax.experimental import pallas as pl
from jax.experimental.pallas import tpu as pltpu
from jax.experimental.pallas import tpu_sc as plsc

C = 19
N = 8 * 384 * 384
B = 1024
NW = 32
CH = N // NW
WBLK = 73728
NHB = (384 * 384) // WBLK


def _softmax_errs_body(lg_ref, lab_ref, errs_ref):
    x = lg_ref[0].reshape(C, WBLK)
    lab = lab_ref[0].reshape(1, WBLK)
    m = jnp.max(x, axis=0, keepdims=True)
    ex = jnp.exp(x - m)
    p = ex / jnp.sum(ex, axis=0, keepdims=True)
    oh = lax.broadcasted_iota(jnp.int32, (C, WBLK), 0) == lab
    errs_ref[...] = jnp.where(oh, 1.0 - p, p)


def _softmax_errs(logits, label):
    hrows = 384 // NHB
    return pl.pallas_call(
        _softmax_errs_body,
        grid=(8, NHB),
        in_specs=[
            pl.BlockSpec((1, C, hrows, 384), lambda n, hb: (n, 0, hb, 0)),
            pl.BlockSpec((1, hrows, 384), lambda n, hb: (n, hb, 0)),
        ],
        out_specs=pl.BlockSpec((C, WBLK), lambda n, hb: (0, n * NHB + hb)),
        out_shape=jax.ShapeDtypeStruct((C, N), jnp.float32),
    )(logits, label)


def _hist_body(errs_hbm, lab_hbm, pcnt_hbm,
               lab_v, err_a, err_b, cnt_v, sem_a, sem_b):
    wid = lax.axis_index("s") * 2 + lax.axis_index("c")
    base = wid * CH
    pltpu.sync_copy(lab_hbm.at[pl.ds(base, CH)], lab_v)
    ones = jnp.full((16,), 1, jnp.int32)
    zeros_i = jnp.zeros((16,), jnp.int32)

    def dma(c, buf, sem):
        return pltpu.make_async_copy(
            errs_hbm.at[pl.ds(c, 1), pl.ds(base, CH)], buf, sem)

    def work(c, buf):
        @plsc.parallel_loop(0, 2 * B, 16, unroll=8)
        def zero_body(i):
            cnt_v[pl.ds(i, 16)] = zeros_i

        @plsc.parallel_loop(0, CH, 16, unroll=16)
        def pix_body(i):
            e = buf[0, pl.ds(i, 16)]
            lb = lab_v[pl.ds(i, 16)]
            b = jnp.minimum((e * float(B)).astype(jnp.int32), B - 1)
            idx = b + jnp.where(lb == c, B, 0)
            plsc.addupdate_scatter(cnt_v, [idx], ones)

        off = (c * NW + wid) * (2 * B)
        pltpu.sync_copy(cnt_v, pcnt_hbm.at[pl.ds(off, 2 * B)])

    dma(0, err_a, sem_a).start()

    def pair_body(j, _):
        c0 = 2 * j
        dma(c0, err_a, sem_a).wait()
        dma(c0 + 1, err_b, sem_b).start()
        work(c0, err_a)
        dma(c0 + 1, err_b, sem_b).wait()

        @pl.when(c0 + 2 < C)
        def _():
            dma(c0 + 2, err_a, sem_a).start()

        work(c0 + 1, err_b)
        return 0

    lax.fori_loop(0, C // 2, pair_body, 0)
    dma(C - 1, err_a, sem_a).wait()
    work(C - 1, err_a)


def _histograms(errs2d, lab_flat):
    mesh = plsc.VectorSubcoreMesh(core_axis_name="c", subcore_axis_name="s", num_cores=2, num_subcores=16)
    return pl.kernel(
        _hist_body,
        out_type=jax.ShapeDtypeStruct((C * NW * 2 * B,), jnp.int32),
        mesh=mesh,
        scratch_types=[
            pltpu.VMEM((CH,), jnp.int32),
            pltpu.VMEM((1, CH), jnp.float32),
            pltpu.VMEM((1, CH), jnp.float32),
            pltpu.VMEM((2 * B,), jnp.int32),
            pltpu.SemaphoreType.DMA,
            pltpu.SemaphoreType.DMA,
        ],
        compiler_params=pltpu.CompilerParams(needs_layout_passes=False),
    )(errs2d, lab_flat)


def _loss_body(pcnt_hbm, out_hbm,
               acc_c, buf_c, out_v):
    wid = lax.axis_index("s") * 2 + lax.axis_index("c")

    @pl.when(wid < C)
    def _():
        c = wid
        pltpu.sync_copy(pcnt_hbm.at[pl.ds(c * NW * 2 * B, NW * 2 * B)], buf_c)

        def wsum_body(i, _):
            sl = pl.ds(i * 16, 16)
            v = buf_c[sl]

            def inner(w, v):
                return v + buf_c[pl.ds(w * 2 * B + i * 16, 16)]
            acc_c[sl] = lax.fori_loop(1, NW, inner, v)
            return 0
        lax.fori_loop(0, (2 * B) // 16, wsum_body, 0)

        def pos_body(j, p):
            return p + jnp.sum(acc_c[pl.ds(B + j * 16, 16)])
        P = lax.fori_loop(0, B // 16, pos_body, jnp.int32(0))
        Pf = P.astype(jnp.float32)

        def jacc(m, k):
            den = Pf + m - k
            return jnp.where(den > 0.0, 1.0 - (Pf - k) / jnp.where(den > 0.0, den, 1.0), 0.0)

        lanes_f = lax.iota(jnp.int32, 16).astype(jnp.float32)

        def blk_body(j, carry):
            m0, k0, acc = carry
            b0 = B - 16 * (j + 1)
            cn = lax.rev(acc_c[pl.ds(b0, 16)], (0,))
            cp = lax.rev(acc_c[pl.ds(B + b0, 16)], (0,))
            n = cn + cp
            mc = jnp.cumsum(n) + m0
            kc = jnp.cumsum(cp) + k0
            mcf = mc.astype(jnp.float32)
            kcf = kc.astype(jnp.float32)
            nf = n.astype(jnp.float32)
            kf = cp.astype(jnp.float32)
            j_after = jacc(mcf, kcf)
            j_before = jacc(mcf - nf, kcf - kf)
            mean = (jnp.float32(b0) + 15.5 - lanes_f) * jnp.float32(1.0 / B)
            acc = acc + jnp.sum(mean * (j_after - j_before))
            return (m0 + jnp.sum(n), k0 + jnp.sum(cp), acc)

        _, _, loss = lax.fori_loop(
            0, B // 16, blk_body,
            (jnp.int32(0), jnp.int32(0), jnp.float32(0.0)))

        out_v[...] = jnp.broadcast_to(loss, (16,))
        pltpu.sync_copy(out_v, out_hbm.at[pl.ds(c * 16, 16)])


def _losses(pcnt):
    mesh = plsc.VectorSubcoreMesh(core_axis_name="c", subcore_axis_name="s", num_cores=2, num_subcores=16)
    return pl.kernel(
        _loss_body,
        out_type=jax.ShapeDtypeStruct((C * 16,), jnp.float32),
        mesh=mesh,
        scratch_types=[
            pltpu.VMEM((2 * B,), jnp.int32),
            pltpu.VMEM((NW * 2 * B,), jnp.int32),
            pltpu.VMEM((16,), jnp.float32),
        ],
        compiler_params=pltpu.CompilerParams(needs_layout_passes=False),
    )(pcnt)


def kernel(logits, label):
    errs = _softmax_errs(logits, label)
    lab_flat = label.reshape(-1)
    pcnt = _histograms(errs, lab_flat)
    lossv = _losses(pcnt)
    loss = jnp.mean(lossv.reshape(C, 16)[:, 0])
    return (loss, errs)
